# baseline (speedup 1.0000x reference)
import functools

import jax
import jax.numpy as jnp
from jax import lax
from jax.experimental import pallas as pl
from jax.experimental.pallas import tpu as pltpu
from jax.experimental.pallas import tpu_sc as plsc

NP = 10240
DP = 40
NTILES = 32
ROWS_PER_TILE = NP // 16

_mesh = plsc.VectorSubcoreMesh(core_axis_name="c", subcore_axis_name="s")


def _deg_kernel_factory(ept):
  @functools.partial(
      pl.kernel,
      out_type=jax.ShapeDtypeStruct((NTILES, NP), jnp.float32),
      mesh=_mesh,
      scratch_types=[
          pltpu.VMEM((ept,), jnp.int32),
          pltpu.VMEM((NP,), jnp.float32),
      ],
      compiler_params=pltpu.CompilerParams(needs_layout_passes=False,
                                           use_tc_tiling_on_sc=False),
  )
  def deg_kernel(edge_hbm, out_hbm, idx_v, deg_v):
    c = lax.axis_index("c")
    s = lax.axis_index("s")
    wid = s * 2 + c

    def zero_body(i, carry):
      for u in range(8):
        deg_v[pl.ds(i * 128 + u * 16, 16)] = jnp.zeros((16,), jnp.float32)
      return carry

    lax.fori_loop(0, NP // 128, zero_body, 0)

    pltpu.sync_copy(edge_hbm.at[1, pl.ds(wid * ept, ept)], idx_v)

    ones = jnp.ones((16,), jnp.float32)

    def body(i, carry):
      for u in range(8):
        idx = idx_v[pl.ds(i * 128 + u * 16, 16)]
        plsc.addupdate_scatter(deg_v, [idx], ones)
      return carry

    lax.fori_loop(0, ept // 128, body, 0)
    for u in range((ept % 128) // 16):
      idx = idx_v[pl.ds((ept // 128) * 128 + u * 16, 16)]
      plsc.addupdate_scatter(deg_v, [idx], ones)

    pltpu.sync_copy(deg_v, out_hbm.at[wid])

  return deg_kernel


NBUF = 1
CH = 400


def _prop_kernel_factory(ept):
  n_chunks = ept // CH
  n_groups = n_chunks // NBUF

  @functools.partial(
      pl.kernel,
      out_type=jax.ShapeDtypeStruct((2, NP, DP), jnp.float32),
      mesh=_mesh,
      scratch_types=[
          pltpu.VMEM((ept,), jnp.int32),
          pltpu.VMEM((ept,), jnp.int32),
          pltpu.VMEM((2, NBUF, CH, DP), jnp.float32),
          pltpu.VMEM_SHARED((NP, DP), jnp.float32),
          pltpu.VMEM_SHARED((NP, DP), jnp.float32),
          pltpu.SemaphoreType.DMA,
          pltpu.SemaphoreType.DMA,
      ],
      compiler_params=pltpu.CompilerParams(use_tc_tiling_on_sc=False),
  )
  def prop_kernel(g_hbm, edge_hbm, zeros_hbm, out_hbm, sidx_v, didx_v,
                  rows_v, acc_sh, gtab_sh, gsem, ssem):
    c = lax.axis_index("c")
    s = lax.axis_index("s")
    wid = s * 2 + c

    @pl.when(c == 0)
    def _():
      pltpu.sync_copy(g_hbm.at[pl.ds(s * ROWS_PER_TILE, ROWS_PER_TILE)],
                      acc_sh.at[pl.ds(s * ROWS_PER_TILE, ROWS_PER_TILE)])

    @pl.when(c == 1)
    def _():
      pltpu.sync_copy(zeros_hbm.at[pl.ds(s * ROWS_PER_TILE, ROWS_PER_TILE)],
                      acc_sh.at[pl.ds(s * ROWS_PER_TILE, ROWS_PER_TILE)])

    pltpu.sync_copy(edge_hbm.at[0, pl.ds(wid * ept, ept)], sidx_v)
    pltpu.sync_copy(edge_hbm.at[1, pl.ds(wid * ept, ept)], didx_v)
    pltpu.sync_copy(g_hbm.at[pl.ds(s * ROWS_PER_TILE, ROWS_PER_TILE)],
                    gtab_sh.at[pl.ds(s * ROWS_PER_TILE, ROWS_PER_TILE)])

    plsc.subcore_barrier()

    def gather_start(g, p, k):
      pltpu.async_copy(gtab_sh.at[sidx_v.at[pl.ds((g * NBUF + k) * CH, CH)]],
                       rows_v.at[p, k], gsem)

    def gather_wait(g, p, k):
      pltpu.make_async_copy(
          gtab_sh.at[sidx_v.at[pl.ds((g * NBUF + k) * CH, CH)]],
          rows_v.at[p, k], gsem).wait()

    def scatter_start(g, p, k):
      pltpu.async_copy(rows_v.at[p, k],
                       acc_sh.at[didx_v.at[pl.ds((g * NBUF + k) * CH, CH)]],
                       ssem, add=True)

    def scatter_wait(g, p, k):
      pltpu.make_async_copy(
          rows_v.at[p, k],
          acc_sh.at[didx_v.at[pl.ds((g * NBUF + k) * CH, CH)]], ssem).wait()

    for k in range(NBUF):
      gather_start(0, 0, k)

    def group_body(g, carry):
      p = lax.rem(g, 2)
      for k in range(NBUF):
        gather_wait(g, p, k)

      @pl.when(g < n_groups - 1)
      def _():
        for k in range(NBUF):
          gather_start(g + 1, 1 - p, k)

      for k in range(NBUF):
        scatter_start(g, p, k)
      for k in range(NBUF):
        scatter_wait(g, p, k)
      return carry

    lax.fori_loop(0, n_groups, group_body, 0)

    plsc.subcore_barrier()

    pltpu.sync_copy(acc_sh.at[pl.ds(s * ROWS_PER_TILE, ROWS_PER_TILE)],
                    out_hbm.at[c, pl.ds(s * ROWS_PER_TILE, ROWS_PER_TILE)])

  return prop_kernel


def _mm_body(x_ref, w_ref, h_ref):
  h_ref[...] = jnp.dot(x_ref[...], w_ref[...],
                       preferred_element_type=jnp.float32)


def _scale_body(cnt_ref, h_ref, g_ref, dinv_ref):
  deg = jnp.sum(cnt_ref[...], axis=0) + 1.0
  dinv = lax.rsqrt(deg)
  g_ref[...] = h_ref[...] * dinv[:, None]
  dinv_ref[...] = dinv[None, :]


def _fin_body(s_ref, dinv_ref, b_ref, o_ref):
  t = s_ref[0] + s_ref[1]
  o_ref[...] = t * dinv_ref[0][:, None] + b_ref[...]


def kernel(x, edge_index, W, b):
  n = x.shape[0]
  e = edge_index.shape[1]
  d_out = W.shape[1]

  ei = edge_index.astype(jnp.int32)
  ept = e // NTILES

  Wp = jnp.pad(W, ((0, 0), (0, DP - d_out)))
  b2 = jnp.pad(b, (0, DP - d_out))[None, :]

  cnt = _deg_kernel_factory(ept)(ei)

  blk = 1024
  h = pl.pallas_call(
      _mm_body,
      grid=(NP // blk,),
      in_specs=[
          pl.BlockSpec((blk, 128), lambda i: (i, 0)),
          pl.BlockSpec((128, DP), lambda i: (0, 0)),
      ],
      out_specs=pl.BlockSpec((blk, DP), lambda i: (i, 0)),
      out_shape=jax.ShapeDtypeStruct((NP, DP), jnp.float32),
  )(x, Wp)
  g, dinv = pl.pallas_call(
      _scale_body,
      grid=(NP // blk,),
      in_specs=[
          pl.BlockSpec((NTILES, blk), lambda i: (0, i)),
          pl.BlockSpec((blk, DP), lambda i: (i, 0)),
      ],
      out_specs=[
          pl.BlockSpec((blk, DP), lambda i: (i, 0)),
          pl.BlockSpec((1, blk), lambda i: (0, i)),
      ],
      out_shape=[
          jax.ShapeDtypeStruct((NP, DP), jnp.float32),
          jax.ShapeDtypeStruct((1, NP), jnp.float32),
      ],
  )(cnt, h)

  zeros = jnp.zeros((NP, DP), jnp.float32)
  S = _prop_kernel_factory(ept)(g, ei, zeros)

  outp = pl.pallas_call(
      _fin_body,
      grid=(NP // blk,),
      in_specs=[
          pl.BlockSpec((2, blk, DP), lambda i: (0, i, 0)),
          pl.BlockSpec((1, blk), lambda i: (0, i)),
          pl.BlockSpec((1, DP), lambda i: (0, 0)),
      ],
      out_specs=pl.BlockSpec((blk, DP), lambda i: (i, 0)),
      out_shape=jax.ShapeDtypeStruct((n, DP), jnp.float32),
  )(S, dinv, b2)

  return outp[:, :d_out]


# speedup vs baseline: 1.0005x; 1.0005x over previous
import functools

import jax
import jax.numpy as jnp
from jax import lax
from jax.experimental import pallas as pl
from jax.experimental.pallas import tpu as pltpu
from jax.experimental.pallas import tpu_sc as plsc

NP = 10240
DP = 40
NTILES = 32
ROWS_PER_TILE = NP // 16

_mesh = plsc.VectorSubcoreMesh(core_axis_name="c", subcore_axis_name="s")


def _deg_kernel_factory(ept):
  @functools.partial(
      pl.kernel,
      out_type=jax.ShapeDtypeStruct((NTILES, NP), jnp.float32),
      mesh=_mesh,
      scratch_types=[
          pltpu.VMEM((ept,), jnp.int32),
          pltpu.VMEM((NP,), jnp.float32),
      ],
      compiler_params=pltpu.CompilerParams(needs_layout_passes=False,
                                           use_tc_tiling_on_sc=False),
  )
  def deg_kernel(edge_hbm, out_hbm, idx_v, deg_v):
    c = lax.axis_index("c")
    s = lax.axis_index("s")
    wid = s * 2 + c

    def zero_body(i, carry):
      for u in range(8):
        deg_v[pl.ds(i * 128 + u * 16, 16)] = jnp.zeros((16,), jnp.float32)
      return carry

    lax.fori_loop(0, NP // 128, zero_body, 0)

    pltpu.sync_copy(edge_hbm.at[1, pl.ds(wid * ept, ept)], idx_v)

    ones = jnp.ones((16,), jnp.float32)

    def body(i, carry):
      for u in range(8):
        idx = idx_v[pl.ds(i * 128 + u * 16, 16)]
        plsc.addupdate_scatter(deg_v, [idx], ones)
      return carry

    lax.fori_loop(0, ept // 128, body, 0)
    for u in range((ept % 128) // 16):
      idx = idx_v[pl.ds((ept // 128) * 128 + u * 16, 16)]
      plsc.addupdate_scatter(deg_v, [idx], ones)

    pltpu.sync_copy(deg_v, out_hbm.at[wid])

  return deg_kernel


NBUF = 2
CH = 200


def _prop_kernel_factory(ept):
  n_chunks = ept // CH
  n_groups = n_chunks // NBUF

  @functools.partial(
      pl.kernel,
      out_type=jax.ShapeDtypeStruct((2, NP, DP), jnp.float32),
      mesh=_mesh,
      scratch_types=[
          pltpu.VMEM((ept,), jnp.int32),
          pltpu.VMEM((ept,), jnp.int32),
          pltpu.VMEM((2, NBUF, CH, DP), jnp.float32),
          pltpu.VMEM_SHARED((NP, DP), jnp.float32),
          pltpu.VMEM_SHARED((NP, DP), jnp.float32),
          pltpu.SemaphoreType.DMA,
          pltpu.SemaphoreType.DMA,
      ],
      compiler_params=pltpu.CompilerParams(use_tc_tiling_on_sc=False),
  )
  def prop_kernel(g_hbm, edge_hbm, zeros_hbm, out_hbm, sidx_v, didx_v,
                  rows_v, acc_sh, gtab_sh, gsem, ssem):
    c = lax.axis_index("c")
    s = lax.axis_index("s")
    wid = s * 2 + c

    @pl.when(c == 0)
    def _():
      pltpu.sync_copy(g_hbm.at[pl.ds(s * ROWS_PER_TILE, ROWS_PER_TILE)],
                      acc_sh.at[pl.ds(s * ROWS_PER_TILE, ROWS_PER_TILE)])

    @pl.when(c == 1)
    def _():
      pltpu.sync_copy(zeros_hbm.at[pl.ds(s * ROWS_PER_TILE, ROWS_PER_TILE)],
                      acc_sh.at[pl.ds(s * ROWS_PER_TILE, ROWS_PER_TILE)])

    pltpu.sync_copy(edge_hbm.at[0, pl.ds(wid * ept, ept)], sidx_v)
    pltpu.sync_copy(edge_hbm.at[1, pl.ds(wid * ept, ept)], didx_v)
    pltpu.sync_copy(g_hbm.at[pl.ds(s * ROWS_PER_TILE, ROWS_PER_TILE)],
                    gtab_sh.at[pl.ds(s * ROWS_PER_TILE, ROWS_PER_TILE)])

    plsc.subcore_barrier()

    def gather_start(g, p, k):
      pltpu.async_copy(gtab_sh.at[sidx_v.at[pl.ds((g * NBUF + k) * CH, CH)]],
                       rows_v.at[p, k], gsem)

    def gather_wait(g, p, k):
      pltpu.make_async_copy(
          gtab_sh.at[sidx_v.at[pl.ds((g * NBUF + k) * CH, CH)]],
          rows_v.at[p, k], gsem).wait()

    def scatter_start(g, p, k):
      pltpu.async_copy(rows_v.at[p, k],
                       acc_sh.at[didx_v.at[pl.ds((g * NBUF + k) * CH, CH)]],
                       ssem, add=True)

    def scatter_wait(g, p, k):
      pltpu.make_async_copy(
          rows_v.at[p, k],
          acc_sh.at[didx_v.at[pl.ds((g * NBUF + k) * CH, CH)]], ssem).wait()

    for k in range(NBUF):
      gather_start(0, 0, k)

    def group_body(g, carry):
      p = lax.rem(g, 2)
      for k in range(NBUF):
        gather_wait(g, p, k)

      @pl.when(g < n_groups - 1)
      def _():
        for k in range(NBUF):
          gather_start(g + 1, 1 - p, k)

      for k in range(NBUF):
        scatter_start(g, p, k)
      for k in range(NBUF):
        scatter_wait(g, p, k)
      return carry

    lax.fori_loop(0, n_groups, group_body, 0)

    plsc.subcore_barrier()

    pltpu.sync_copy(acc_sh.at[pl.ds(s * ROWS_PER_TILE, ROWS_PER_TILE)],
                    out_hbm.at[c, pl.ds(s * ROWS_PER_TILE, ROWS_PER_TILE)])

  return prop_kernel


def _mm_body(x_ref, w_ref, h_ref):
  h_ref[...] = jnp.dot(x_ref[...], w_ref[...],
                       preferred_element_type=jnp.float32)


def _scale_body(cnt_ref, h_ref, g_ref, dinv_ref):
  deg = jnp.sum(cnt_ref[...], axis=0) + 1.0
  dinv = lax.rsqrt(deg)
  g_ref[...] = h_ref[...] * dinv[:, None]
  dinv_ref[...] = dinv[None, :]


def _fin_body(s_ref, dinv_ref, b_ref, o_ref):
  t = s_ref[0] + s_ref[1]
  o_ref[...] = t * dinv_ref[0][:, None] + b_ref[...]


def kernel(x, edge_index, W, b):
  n = x.shape[0]
  e = edge_index.shape[1]
  d_out = W.shape[1]

  ei = edge_index.astype(jnp.int32)
  ept = e // NTILES

  Wp = jnp.pad(W, ((0, 0), (0, DP - d_out)))
  b2 = jnp.pad(b, (0, DP - d_out))[None, :]

  cnt = _deg_kernel_factory(ept)(ei)

  blk = 1024
  h = pl.pallas_call(
      _mm_body,
      grid=(NP // blk,),
      in_specs=[
          pl.BlockSpec((blk, 128), lambda i: (i, 0)),
          pl.BlockSpec((128, DP), lambda i: (0, 0)),
      ],
      out_specs=pl.BlockSpec((blk, DP), lambda i: (i, 0)),
      out_shape=jax.ShapeDtypeStruct((NP, DP), jnp.float32),
  )(x, Wp)
  g, dinv = pl.pallas_call(
      _scale_body,
      grid=(NP // blk,),
      in_specs=[
          pl.BlockSpec((NTILES, blk), lambda i: (0, i)),
          pl.BlockSpec((blk, DP), lambda i: (i, 0)),
      ],
      out_specs=[
          pl.BlockSpec((blk, DP), lambda i: (i, 0)),
          pl.BlockSpec((1, blk), lambda i: (0, i)),
      ],
      out_shape=[
          jax.ShapeDtypeStruct((NP, DP), jnp.float32),
          jax.ShapeDtypeStruct((1, NP), jnp.float32),
      ],
  )(cnt, h)

  zeros = jnp.zeros((NP, DP), jnp.float32)
  S = _prop_kernel_factory(ept)(g, ei, zeros)

  outp = pl.pallas_call(
      _fin_body,
      grid=(NP // blk,),
      in_specs=[
          pl.BlockSpec((2, blk, DP), lambda i: (0, i, 0)),
          pl.BlockSpec((1, blk), lambda i: (0, i)),
          pl.BlockSpec((1, DP), lambda i: (0, 0)),
      ],
      out_specs=pl.BlockSpec((blk, DP), lambda i: (i, 0)),
      out_shape=jax.ShapeDtypeStruct((n, DP), jnp.float32),
  )(S, dinv, b2)

  return outp[:, :d_out]


# speedup vs baseline: 1.0300x; 1.0295x over previous
import functools

import jax
import jax.numpy as jnp
from jax import lax
from jax.experimental import pallas as pl
from jax.experimental.pallas import tpu as pltpu
from jax.experimental.pallas import tpu_sc as plsc

NP = 10240
DP = 40
NTILES = 32
ROWS_PER_TILE = NP // 16

_mesh = plsc.VectorSubcoreMesh(core_axis_name="c", subcore_axis_name="s")


def _deg_kernel_factory(ept):
  @functools.partial(
      pl.kernel,
      out_type=jax.ShapeDtypeStruct((NTILES, NP), jnp.float32),
      mesh=_mesh,
      scratch_types=[
          pltpu.VMEM((ept,), jnp.int32),
          pltpu.VMEM((NP,), jnp.float32),
      ],
      compiler_params=pltpu.CompilerParams(needs_layout_passes=False,
                                           use_tc_tiling_on_sc=False),
  )
  def deg_kernel(edge_hbm, out_hbm, idx_v, deg_v):
    c = lax.axis_index("c")
    s = lax.axis_index("s")
    wid = s * 2 + c

    def zero_body(i, carry):
      for u in range(8):
        deg_v[pl.ds(i * 128 + u * 16, 16)] = jnp.zeros((16,), jnp.float32)
      return carry

    lax.fori_loop(0, NP // 128, zero_body, 0)

    pltpu.sync_copy(edge_hbm.at[1, pl.ds(wid * ept, ept)], idx_v)

    ones = jnp.ones((16,), jnp.float32)

    def body(i, carry):
      for u in range(8):
        idx = idx_v[pl.ds(i * 128 + u * 16, 16)]
        plsc.addupdate_scatter(deg_v, [idx], ones)
      return carry

    lax.fori_loop(0, ept // 128, body, 0)
    for u in range((ept % 128) // 16):
      idx = idx_v[pl.ds((ept // 128) * 128 + u * 16, 16)]
      plsc.addupdate_scatter(deg_v, [idx], ones)

    pltpu.sync_copy(deg_v, out_hbm.at[wid])

  return deg_kernel


NBUF = 2
CH = 200


def _prop_kernel_factory(ept):
  n_chunks = ept // CH
  n_groups = n_chunks // NBUF

  @functools.partial(
      pl.kernel,
      out_type=jax.ShapeDtypeStruct((2, NP, DP), jnp.float32),
      mesh=_mesh,
      scratch_types=[
          pltpu.VMEM((ept,), jnp.int32),
          pltpu.VMEM((ept,), jnp.int32),
          pltpu.VMEM((2, NBUF, CH, DP), jnp.float32),
          pltpu.VMEM_SHARED((NP, DP), jnp.float32),
          pltpu.VMEM_SHARED((NP, DP), jnp.float32),
          pltpu.SemaphoreType.DMA,
          pltpu.SemaphoreType.DMA,
      ],
      compiler_params=pltpu.CompilerParams(use_tc_tiling_on_sc=False),
  )
  def prop_kernel(g_hbm, edge_hbm, zeros_hbm, out_hbm, sidx_v, didx_v,
                  rows_v, acc_sh, gtab_sh, gsem, ssem):
    c = lax.axis_index("c")
    s = lax.axis_index("s")
    wid = s * 2 + c

    @pl.when(c == 0)
    def _():
      pltpu.sync_copy(g_hbm.at[pl.ds(s * ROWS_PER_TILE, ROWS_PER_TILE)],
                      acc_sh.at[pl.ds(s * ROWS_PER_TILE, ROWS_PER_TILE)])

    @pl.when(c == 1)
    def _():
      pltpu.sync_copy(zeros_hbm.at[pl.ds(s * ROWS_PER_TILE, ROWS_PER_TILE)],
                      acc_sh.at[pl.ds(s * ROWS_PER_TILE, ROWS_PER_TILE)])

    pltpu.sync_copy(edge_hbm.at[0, pl.ds(wid * ept, ept)], sidx_v)
    pltpu.sync_copy(edge_hbm.at[1, pl.ds(wid * ept, ept)], didx_v)
    pltpu.sync_copy(g_hbm.at[pl.ds(s * ROWS_PER_TILE, ROWS_PER_TILE)],
                    gtab_sh.at[pl.ds(s * ROWS_PER_TILE, ROWS_PER_TILE)])

    plsc.subcore_barrier()

    def gather_start(g, p, k):
      pltpu.async_copy(gtab_sh.at[sidx_v.at[pl.ds((g * NBUF + k) * CH, CH)]],
                       rows_v.at[p, k], gsem)

    def gather_wait(g, p, k):
      pltpu.make_async_copy(
          gtab_sh.at[sidx_v.at[pl.ds((g * NBUF + k) * CH, CH)]],
          rows_v.at[p, k], gsem).wait()

    def scatter_start(g, p, k):
      pltpu.async_copy(rows_v.at[p, k],
                       acc_sh.at[didx_v.at[pl.ds((g * NBUF + k) * CH, CH)]],
                       ssem, add=True)

    def scatter_wait(g, p, k):
      pltpu.make_async_copy(
          rows_v.at[p, k],
          acc_sh.at[didx_v.at[pl.ds((g * NBUF + k) * CH, CH)]], ssem).wait()

    for k in range(NBUF):
      gather_start(0, 0, k)

    def group_body(g, carry):
      p = lax.rem(g, 2)
      for k in range(NBUF):
        gather_wait(g, p, k)

      @pl.when(g < n_groups - 1)
      def _():
        for k in range(NBUF):
          gather_start(g + 1, 1 - p, k)

      for k in range(NBUF):
        scatter_start(g, p, k)
      for k in range(NBUF):
        scatter_wait(g, p, k)
      return carry

    lax.fori_loop(0, n_groups, group_body, 0)

    plsc.subcore_barrier()

    pltpu.sync_copy(acc_sh.at[pl.ds(s * ROWS_PER_TILE, ROWS_PER_TILE)],
                    out_hbm.at[c, pl.ds(s * ROWS_PER_TILE, ROWS_PER_TILE)])

  return prop_kernel


def _mm_body(cnt_ref, x_ref, w_ref, g_ref, dinv_ref):
  deg = jnp.sum(cnt_ref[...], axis=0) + 1.0
  dinv = lax.rsqrt(deg)
  h = jnp.dot(x_ref[...], w_ref[...], preferred_element_type=jnp.float32)
  g_ref[...] = h * dinv[:, None]
  dinv_ref[...] = dinv[None, :]


def _fin_body(s_ref, dinv_ref, b_ref, o_ref):
  t = s_ref[0] + s_ref[1]
  o_ref[...] = t * dinv_ref[0][:, None] + b_ref[...]


def kernel(x, edge_index, W, b):
  n = x.shape[0]
  e = edge_index.shape[1]
  d_out = W.shape[1]

  ei = edge_index.astype(jnp.int32)
  ept = e // NTILES

  Wp = jnp.pad(W, ((0, 0), (0, DP - d_out)))
  b2 = jnp.pad(b, (0, DP - d_out))[None, :]

  cnt = _deg_kernel_factory(ept)(ei)

  blk = 1024
  g, dinv = pl.pallas_call(
      _mm_body,
      grid=(NP // blk,),
      in_specs=[
          pl.BlockSpec((NTILES, blk), lambda i: (0, i)),
          pl.BlockSpec((blk, 128), lambda i: (i, 0)),
          pl.BlockSpec((128, DP), lambda i: (0, 0)),
      ],
      out_specs=[
          pl.BlockSpec((blk, DP), lambda i: (i, 0)),
          pl.BlockSpec((1, blk), lambda i: (0, i)),
      ],
      out_shape=[
          jax.ShapeDtypeStruct((NP, DP), jnp.float32),
          jax.ShapeDtypeStruct((1, NP), jnp.float32),
      ],
  )(cnt, x, Wp)

  zeros = jnp.zeros((NP, DP), jnp.float32)
  S = _prop_kernel_factory(ept)(g, ei, zeros)

  outp = pl.pallas_call(
      _fin_body,
      grid=(NP // blk,),
      in_specs=[
          pl.BlockSpec((2, blk, DP), lambda i: (0, i, 0)),
          pl.BlockSpec((1, blk), lambda i: (0, i)),
          pl.BlockSpec((1, DP), lambda i: (0, 0)),
      ],
      out_specs=pl.BlockSpec((blk, DP), lambda i: (i, 0)),
      out_shape=jax.ShapeDtypeStruct((n, DP), jnp.float32),
  )(S, dinv, b2)

  return outp[:, :d_out]


# speedup vs baseline: 1.0416x; 1.0112x over previous
import functools

import jax
import jax.numpy as jnp
from jax import lax
from jax.experimental import pallas as pl
from jax.experimental.pallas import tpu as pltpu
from jax.experimental.pallas import tpu_sc as plsc

NP = 10240
DP = 40
NTILES = 32
ROWS_PER_TILE = NP // 16

_mesh = plsc.VectorSubcoreMesh(core_axis_name="c", subcore_axis_name="s")


def _deg_kernel_factory(ept):
  @functools.partial(
      pl.kernel,
      out_type=jax.ShapeDtypeStruct((NTILES, NP), jnp.float32),
      mesh=_mesh,
      scratch_types=[
          pltpu.VMEM((ept,), jnp.int32),
          pltpu.VMEM((NP,), jnp.float32),
      ],
      compiler_params=pltpu.CompilerParams(needs_layout_passes=False,
                                           use_tc_tiling_on_sc=False),
  )
  def deg_kernel(edge_hbm, out_hbm, idx_v, deg_v):
    c = lax.axis_index("c")
    s = lax.axis_index("s")
    wid = s * 2 + c

    def zero_body(i, carry):
      for u in range(8):
        deg_v[pl.ds(i * 128 + u * 16, 16)] = jnp.zeros((16,), jnp.float32)
      return carry

    lax.fori_loop(0, NP // 128, zero_body, 0)

    pltpu.sync_copy(edge_hbm.at[1, pl.ds(wid * ept, ept)], idx_v)

    ones = jnp.ones((16,), jnp.float32)

    def body(i, carry):
      for u in range(8):
        idx = idx_v[pl.ds(i * 128 + u * 16, 16)]
        plsc.addupdate_scatter(deg_v, [idx], ones)
      return carry

    lax.fori_loop(0, ept // 128, body, 0)
    for u in range((ept % 128) // 16):
      idx = idx_v[pl.ds((ept // 128) * 128 + u * 16, 16)]
      plsc.addupdate_scatter(deg_v, [idx], ones)

    pltpu.sync_copy(deg_v, out_hbm.at[wid])

  return deg_kernel


NBUF = 2
CH = 200


def _prop_kernel_factory(ept):
  n_chunks = ept // CH
  n_groups = n_chunks // NBUF

  @functools.partial(
      pl.kernel,
      out_type=jax.ShapeDtypeStruct((2, NP, DP), jnp.float32),
      mesh=_mesh,
      scratch_types=[
          pltpu.VMEM((ept,), jnp.int32),
          pltpu.VMEM((ept,), jnp.int32),
          pltpu.VMEM((2, NBUF, CH, DP), jnp.float32),
          pltpu.VMEM_SHARED((NP, DP), jnp.float32),
          pltpu.VMEM_SHARED((NP, DP), jnp.float32),
          pltpu.SemaphoreType.DMA,
          pltpu.SemaphoreType.DMA,
      ],
      compiler_params=pltpu.CompilerParams(use_tc_tiling_on_sc=False),
  )
  def prop_kernel(g_hbm, edge_hbm, zeros_hbm, out_hbm, sidx_v, didx_v,
                  rows_v, acc_sh, gtab_sh, gsem, ssem):
    c = lax.axis_index("c")
    s = lax.axis_index("s")
    wid = s * 2 + c

    rslice = pl.ds(s * ROWS_PER_TILE, ROWS_PER_TILE)

    @pl.when(c == 0)
    def _():
      pltpu.async_copy(g_hbm.at[rslice], acc_sh.at[rslice], gsem)

    @pl.when(c == 1)
    def _():
      pltpu.async_copy(zeros_hbm.at[rslice], acc_sh.at[rslice], gsem)

    pltpu.async_copy(edge_hbm.at[0, pl.ds(wid * ept, ept)], sidx_v, gsem)
    pltpu.async_copy(edge_hbm.at[1, pl.ds(wid * ept, ept)], didx_v, gsem)
    pltpu.async_copy(g_hbm.at[rslice], gtab_sh.at[rslice], gsem)

    pltpu.make_async_copy(zeros_hbm.at[rslice], acc_sh.at[rslice], gsem).wait()
    pltpu.make_async_copy(edge_hbm.at[0, pl.ds(wid * ept, ept)], sidx_v,
                          gsem).wait()
    pltpu.make_async_copy(edge_hbm.at[1, pl.ds(wid * ept, ept)], didx_v,
                          gsem).wait()
    pltpu.make_async_copy(g_hbm.at[rslice], gtab_sh.at[rslice], gsem).wait()

    plsc.subcore_barrier()

    def gather_start(g, p, k):
      pltpu.async_copy(gtab_sh.at[sidx_v.at[pl.ds((g * NBUF + k) * CH, CH)]],
                       rows_v.at[p, k], gsem)

    def gather_wait(g, p, k):
      pltpu.make_async_copy(
          gtab_sh.at[sidx_v.at[pl.ds((g * NBUF + k) * CH, CH)]],
          rows_v.at[p, k], gsem).wait()

    def scatter_start(g, p, k):
      pltpu.async_copy(rows_v.at[p, k],
                       acc_sh.at[didx_v.at[pl.ds((g * NBUF + k) * CH, CH)]],
                       ssem, add=True)

    def scatter_wait(g, p, k):
      pltpu.make_async_copy(
          rows_v.at[p, k],
          acc_sh.at[didx_v.at[pl.ds((g * NBUF + k) * CH, CH)]], ssem).wait()

    for k in range(NBUF):
      gather_start(0, 0, k)

    def group_body(g, carry):
      p = lax.rem(g, 2)
      for k in range(NBUF):
        gather_wait(g, p, k)

      @pl.when(g < n_groups - 1)
      def _():
        for k in range(NBUF):
          gather_start(g + 1, 1 - p, k)

      for k in range(NBUF):
        scatter_start(g, p, k)
      for k in range(NBUF):
        scatter_wait(g, p, k)
      return carry

    lax.fori_loop(0, n_groups, group_body, 0)

    plsc.subcore_barrier()

    pltpu.sync_copy(acc_sh.at[pl.ds(s * ROWS_PER_TILE, ROWS_PER_TILE)],
                    out_hbm.at[c, pl.ds(s * ROWS_PER_TILE, ROWS_PER_TILE)])

  return prop_kernel


def _mm_body(cnt_ref, x_ref, w_ref, g_ref, dinv_ref):
  deg = jnp.sum(cnt_ref[...], axis=0) + 1.0
  dinv = lax.rsqrt(deg)
  h = jnp.dot(x_ref[...], w_ref[...], preferred_element_type=jnp.float32)
  g_ref[...] = h * dinv[:, None]
  dinv_ref[...] = dinv[None, :]


def _fin_body(s_ref, dinv_ref, b_ref, o_ref):
  t = s_ref[0] + s_ref[1]
  o_ref[...] = t * dinv_ref[0][:, None] + b_ref[...]


def kernel(x, edge_index, W, b):
  n = x.shape[0]
  e = edge_index.shape[1]
  d_out = W.shape[1]

  ei = edge_index.astype(jnp.int32)
  ept = e // NTILES

  Wp = jnp.pad(W, ((0, 0), (0, DP - d_out)))
  b2 = jnp.pad(b, (0, DP - d_out))[None, :]

  cnt = _deg_kernel_factory(ept)(ei)

  blk = 1024
  g, dinv = pl.pallas_call(
      _mm_body,
      grid=(NP // blk,),
      in_specs=[
          pl.BlockSpec((NTILES, blk), lambda i: (0, i)),
          pl.BlockSpec((blk, 128), lambda i: (i, 0)),
          pl.BlockSpec((128, DP), lambda i: (0, 0)),
      ],
      out_specs=[
          pl.BlockSpec((blk, DP), lambda i: (i, 0)),
          pl.BlockSpec((1, blk), lambda i: (0, i)),
      ],
      out_shape=[
          jax.ShapeDtypeStruct((NP, DP), jnp.float32),
          jax.ShapeDtypeStruct((1, NP), jnp.float32),
      ],
  )(cnt, x, Wp)

  zeros = jnp.zeros((NP, DP), jnp.float32)
  S = _prop_kernel_factory(ept)(g, ei, zeros)

  outp = pl.pallas_call(
      _fin_body,
      grid=(NP // blk,),
      in_specs=[
          pl.BlockSpec((2, blk, DP), lambda i: (0, i, 0)),
          pl.BlockSpec((1, blk), lambda i: (0, i)),
          pl.BlockSpec((1, DP), lambda i: (0, 0)),
      ],
      out_specs=pl.BlockSpec((blk, DP), lambda i: (i, 0)),
      out_shape=jax.ShapeDtypeStruct((n, DP), jnp.float32),
  )(S, dinv, b2)

  return outp[:, :d_out]
